# baseline (speedup 1.0000x reference)
import functools
import math

import jax
import jax.numpy as jnp
from jax import lax
from jax.experimental import pallas as pl
from jax.experimental.pallas import tpu as pltpu
from jax.experimental.pallas import tpu_sc as plsc

_R_CUT = 0.5
_N_BASIS = 8
_GAMMA = (_N_BASIS / _R_CUT) ** 2
_DC = _R_CUT / (_N_BASIS - 1)
_WCOEF = 2.0 * _GAMMA * _DC
_GK = tuple(
    math.exp(-_GAMMA * (((k * _DC) ** 2) - (((k - 1) * _DC) ** 2)))
    for k in range(1, _N_BASIS)
)
_GE = tuple(math.exp(-_GAMMA * (k * _DC) ** 2) for k in range(_N_BASIS))
_SIN_C = (6.28305409, -41.33112295, 81.36549857, -74.47097755, 32.76890242)
_ENV_C = (0.99999983, -9.86951609, 32.46500653, -42.64258497, 29.42397435,
          -10.57911365)

_NC, _NS = 2, 16
_NW = _NC * _NS
_L = 16
_C = 128


def _sc_body(K, n_full, tail_nodes, t_max,
             pos8, nbrf, params, y_out, yv_out,
             idx_v, rows_v, ctr_v, par_v, y_buf, yv_buf, xe_buf, sem):
    wid = lax.axis_index("s") * _NC + lax.axis_index("c")
    pltpu.sync_copy(params, par_v)
    invb = [par_v[c, :] for c in range(3)]
    ang = [par_v[3 + c, :] for c in range(3)]
    iota = lax.iota(jnp.int32, _L)
    zero_i = jnp.zeros((_L,), jnp.int32)
    one_i = jnp.full((_L,), 1, jnp.int32)

    def do_chunk(node_base, n_nodes):
        ebase = pl.multiple_of(node_base * K, 8)
        n_e = n_nodes * K
        pltpu.sync_copy(nbrf.at[pl.ds(ebase, n_e)], idx_v.at[pl.ds(0, n_e)])
        pltpu.sync_copy(pos8.at[pl.ds(node_base, n_nodes)],
                        ctr_v.at[pl.ds(0, n_nodes)])
        pltpu.async_copy(pos8.at[idx_v.at[pl.ds(0, n_e)]],
                         rows_v.at[pl.ds(0, n_e)], sem).wait()

        def edge_vals(e_idx, ctr):
            e = []
            for c in range(3):
                nc = plsc.load_gather(rows_v, [e_idx, zero_i + c])
                u = (nc - ctr[c]) * invb[c]
                up = u - ((u + 12582912.0) - 12582912.0)
                u2 = up * up
                s = (((_SIN_C[4] * u2 + _SIN_C[3]) * u2 + _SIN_C[2]) * u2
                     + _SIN_C[1]) * u2 + _SIN_C[0]
                e.append(ang[c] * (s * up))
            r = e[0] * e[0] + e[1] * e[1] + e[2] * e[2]
            rmax = jnp.maximum(r, 1e-12)
            bi = plsc.bitcast(rmax, jnp.int32)
            bi = 0x5F3759DF - lax.shift_right_logical(bi, one_i)
            yq = plsc.bitcast(bi, jnp.float32)
            hx = 0.5 * rmax
            for _ in range(2):
                yq = yq * (1.5 - hx * yq * yq)
            dd = rmax * yq
            d2 = jnp.minimum(rmax, _R_CUT * _R_CUT)
            env = ((((_ENV_C[5] * d2 + _ENV_C[4]) * d2 + _ENV_C[3]) * d2
                    + _ENV_C[2]) * d2 + _ENV_C[1]) * d2 + _ENV_C[0]
            env = jnp.where(rmax < _R_CUT * _R_CUT, env, 0.0)
            aenv = jnp.exp(-_GAMMA * rmax) * env
            w1 = jnp.exp(_WCOEF * dd)
            w2 = w1 * w1
            w3 = w2 * w1
            w4 = w2 * w2
            w5 = w4 * w1
            w6 = w4 * w2
            w7 = w4 * w3
            ws = (w1, w2, w3, w4, w5, w6, w7)
            xs = [aenv]
            for kk in range(1, _N_BASIS):
                xs.append((ws[kk - 1] * _GE[kk]) * aenv)
            return xs, e

        for gp in range(n_nodes // (2 * _L)):
            nids = [iota + (2 * gp) * _L, iota + (2 * gp + 1) * _L]
            ctrs = [
                tuple(plsc.load_gather(ctr_v, [nid, zero_i + c])
                      for c in range(3))
                for nid in nids
            ]
            gbases = [nid * K for nid in nids]

            def pass1(j, ya, gbases=gbases, ctrs=ctrs):
                out = []
                for p in range(2):
                    xs, e = edge_vals(gbases[p] + j, ctrs[p])
                    for q in range(_N_BASIS):
                        xe_buf[q, j, p, :] = xs[q]
                    for c in range(3):
                        xe_buf[_N_BASIS + c, j, p, :] = e[c]
                    out.extend(ya[p * _N_BASIS + q] + xs[q]
                               for q in range(_N_BASIS))
                return tuple(out)

            ya0 = tuple(jnp.zeros((_L,), jnp.float32)
                        for _ in range(2 * _N_BASIS))
            ya = plsc.parallel_loop(0, K, carry=ya0)(pass1)

            for p in range(2):
                def pass2(j, va, p=p):
                    xq = [xe_buf[q, j, p, :] for q in range(_N_BASIS)]
                    ec = [xe_buf[_N_BASIS + c, j, p, :] for c in range(3)]
                    return tuple(va[q * 3 + c] + xq[q] * ec[c]
                                 for q in range(_N_BASIS) for c in range(3))

                va0 = tuple(jnp.zeros((_L,), jnp.float32)
                            for _ in range(3 * _N_BASIS))
                va = plsc.parallel_loop(0, K, carry=va0, unroll=2)(pass2)
                for q in range(_N_BASIS):
                    plsc.store_scatter(y_buf, [nids[p], zero_i + q],
                                       ya[p * _N_BASIS + q])
                for q in range(3 * _N_BASIS):
                    plsc.store_scatter(yv_buf, [nids[p], zero_i + q], va[q])

        pltpu.sync_copy(y_buf.at[pl.ds(0, n_nodes)],
                        y_out.at[pl.ds(node_base, n_nodes)])
        pltpu.sync_copy(yv_buf.at[pl.ds(0, n_nodes)],
                        yv_out.at[pl.ds(node_base, n_nodes)])

    def t_body(t, carry):
        c = wid + t * _NW

        @pl.when(c < n_full)
        def _():
            do_chunk(pl.multiple_of(c * _C, _C), _C)

        if tail_nodes:
            @pl.when(c == n_full)
            def _():
                do_chunk(pl.multiple_of(n_full * _C, _C), tail_nodes)

        return carry

    lax.fori_loop(0, t_max, t_body, 0)


@functools.cache
def _build(nodes, K):
    n_full = nodes // _C
    tail_nodes = nodes - n_full * _C
    n_chunks = n_full + (1 if tail_nodes else 0)
    t_max = -(-n_chunks // _NW)
    body = functools.partial(_sc_body, K, n_full, tail_nodes, t_max)
    mesh = plsc.VectorSubcoreMesh(core_axis_name="c", subcore_axis_name="s",
                                  num_cores=_NC, num_subcores=_NS)
    fn = pl.kernel(
        body,
        out_type=(
            jax.ShapeDtypeStruct((nodes, _N_BASIS), jnp.float32),
            jax.ShapeDtypeStruct((nodes, 3 * _N_BASIS), jnp.float32),
        ),
        mesh=mesh,
        scratch_types=[
            pltpu.VMEM((_C * K,), jnp.int32),
            pltpu.VMEM((_C * K, 8), jnp.float32),
            pltpu.VMEM((_C, 8), jnp.float32),
            pltpu.VMEM((8, _L), jnp.float32),
            pltpu.VMEM((_C, _N_BASIS), jnp.float32),
            pltpu.VMEM((_C, 3 * _N_BASIS), jnp.float32),
            pltpu.VMEM((_N_BASIS + 3, K, 2, _L), jnp.float32),
            pltpu.SemaphoreType.DMA,
        ],
        compiler_params=pltpu.CompilerParams(needs_layout_passes=False,
                                             use_tc_tiling_on_sc=False),
    )
    return fn


def kernel(pos, box, neighbors):
    batch, nodes, _ = pos.shape
    K = neighbors.shape[1]
    fn = _build(nodes, K)
    p = pos.reshape(nodes, 3)
    pos8 = jnp.pad(p, ((0, 0), (0, 5)))
    nbrf = neighbors.reshape(-1)
    inv_box = (1.0 / box).astype(jnp.float32)
    ang = (box / (2.0 * math.pi)).astype(jnp.float32)
    params = jnp.concatenate([
        jnp.repeat(inv_box[:, None], _L, axis=1),
        jnp.repeat(ang[:, None], _L, axis=1),
        jnp.zeros((2, _L), jnp.float32),
    ], axis=0)
    y_flat, yv_flat = fn(pos8, nbrf, params)
    y = y_flat.reshape(batch, nodes, _N_BASIS)
    yv = yv_flat.reshape(batch, nodes, _N_BASIS, 3)
    return (y, yv)


# speedup vs baseline: 1.2158x; 1.2158x over previous
import functools
import math

import jax
import jax.numpy as jnp
from jax import lax
from jax.experimental import pallas as pl
from jax.experimental.pallas import tpu as pltpu
from jax.experimental.pallas import tpu_sc as plsc

_R_CUT = 0.5
_N_BASIS = 8
_GAMMA = (_N_BASIS / _R_CUT) ** 2
_DC = _R_CUT / (_N_BASIS - 1)
_WCOEF = 2.0 * _GAMMA * _DC
_GE = tuple(math.exp(-_GAMMA * (k * _DC) ** 2) for k in range(_N_BASIS))
_SIN_C = (6.28305409, -41.33112295, 81.36549857, -74.47097755, 32.76890242)
_ENV_C = (0.99999983, -9.86951609, 32.46500653, -42.64258497, 29.42397435,
          -10.57911365)

_NC, _NS = 2, 16
_NW = _NC * _NS
_L = 16
_C = 128


def _sc_body(K, n_full, tail_nodes, tt_max,
             pos8, nbrf, params, y_out, yv_out,
             idx_v0, idx_v1, rows_v0, rows_v1, ctr_v0, ctr_v1, par_v,
             y_buf, yv_buf, xe_buf,
             gsem0, gsem1, isem0, isem1, tsem):
    wid = lax.axis_index("s") * _NC + lax.axis_index("c")
    pltpu.sync_copy(params, par_v)
    invb = [par_v[c, :] for c in range(3)]
    ang = [par_v[3 + c, :] for c in range(3)]
    iota = lax.iota(jnp.int32, _L)
    zero_i = jnp.zeros((_L,), jnp.int32)
    one_i = jnp.full((_L,), 1, jnp.int32)
    idx_b = (idx_v0, idx_v1)
    rows_b = (rows_v0, rows_v1)
    ctr_b = (ctr_v0, ctr_v1)
    gsem = (gsem0, gsem1)
    isem = (isem0, isem1)
    CK = _C * K

    def compute_chunk(rows_v, ctr_v, node_base, n_nodes):
        def edge_vals(e_idx, ctr):
            e = []
            for c in range(3):
                nc = plsc.load_gather(rows_v, [e_idx, zero_i + c])
                u = (nc - ctr[c]) * invb[c]
                up = u - ((u + 12582912.0) - 12582912.0)
                u2 = up * up
                s = (((_SIN_C[4] * u2 + _SIN_C[3]) * u2 + _SIN_C[2]) * u2
                     + _SIN_C[1]) * u2 + _SIN_C[0]
                e.append(ang[c] * (s * up))
            r = e[0] * e[0] + e[1] * e[1] + e[2] * e[2]
            rmax = jnp.maximum(r, 1e-12)
            bi = plsc.bitcast(rmax, jnp.int32)
            bi = 0x5F3759DF - lax.shift_right_logical(bi, one_i)
            yq = plsc.bitcast(bi, jnp.float32)
            hx = 0.5 * rmax
            for _ in range(2):
                yq = yq * (1.5 - hx * yq * yq)
            dd = rmax * yq
            d2 = jnp.minimum(rmax, _R_CUT * _R_CUT)
            env = ((((_ENV_C[5] * d2 + _ENV_C[4]) * d2 + _ENV_C[3]) * d2
                    + _ENV_C[2]) * d2 + _ENV_C[1]) * d2 + _ENV_C[0]
            env = jnp.where(rmax < _R_CUT * _R_CUT, env, 0.0)
            aenv = jnp.exp(-_GAMMA * rmax) * env
            w1 = jnp.exp(_WCOEF * dd)
            w2 = w1 * w1
            w3 = w2 * w1
            w4 = w2 * w2
            ws = (w1, w2, w3, w4, w4 * w1, w4 * w2, w4 * w3)
            xs = [aenv]
            for kk in range(1, _N_BASIS):
                xs.append((ws[kk - 1] * _GE[kk]) * aenv)
            return xs, e

        for g in range(n_nodes // _L):
            nid = iota + g * _L
            ctr = tuple(plsc.load_gather(ctr_v, [nid, zero_i + c])
                        for c in range(3))
            gbase = nid * K

            def pass1(j, ya, gbase=gbase, ctr=ctr):
                xs, e = edge_vals(gbase + j, ctr)
                for q in range(_N_BASIS):
                    xe_buf[q, j, :] = xs[q]
                for c in range(3):
                    xe_buf[_N_BASIS + c, j, :] = e[c]
                return tuple(ya[q] + xs[q] for q in range(_N_BASIS))

            ya0 = tuple(jnp.zeros((_L,), jnp.float32) for _ in range(_N_BASIS))
            ya = plsc.parallel_loop(0, K, carry=ya0, unroll=2)(pass1)

            def pass2(j, va):
                xq = [xe_buf[q, j, :] for q in range(_N_BASIS)]
                ec = [xe_buf[_N_BASIS + c, j, :] for c in range(3)]
                return tuple(va[q * 3 + c] + xq[q] * ec[c]
                             for q in range(_N_BASIS) for c in range(3))

            va0 = tuple(jnp.zeros((_L,), jnp.float32)
                        for _ in range(3 * _N_BASIS))
            va = plsc.parallel_loop(0, K, carry=va0, unroll=2)(pass2)
            for q in range(_N_BASIS):
                plsc.store_scatter(y_buf, [nid, zero_i + q], ya[q])
            for q in range(3 * _N_BASIS):
                plsc.store_scatter(yv_buf, [nid, zero_i + q], va[q])

        pltpu.sync_copy(y_buf.at[pl.ds(0, n_nodes)],
                        y_out.at[pl.ds(node_base, n_nodes)])
        pltpu.sync_copy(yv_buf.at[pl.ds(0, n_nodes)],
                        yv_out.at[pl.ds(node_base, n_nodes)])

    c0 = wid

    @pl.when(c0 < n_full)
    def _():
        base = pl.multiple_of(c0 * CK, 8)
        pltpu.sync_copy(nbrf.at[pl.ds(base, CK)], idx_v0)
        pltpu.async_copy(pos8.at[idx_v0], rows_v0, gsem0)
        pltpu.async_copy(pos8.at[pl.ds(pl.multiple_of(c0 * _C, _C), _C)],
                         ctr_v0, gsem0)

    c1 = wid + _NW

    @pl.when(c1 < n_full)
    def _():
        base = pl.multiple_of(c1 * CK, 8)
        pltpu.async_copy(nbrf.at[pl.ds(base, CK)], idx_v1, isem1)

    def tt_body(tt, carry):
        for b in range(2):
            t_eff = 2 * tt + b
            c = wid + t_eff * _NW
            cn = c + _NW
            c2 = c + 2 * _NW

            @pl.when(c < n_full)
            def _(c=c, cn=cn, c2=c2, b=b):
                nb = 1 - b
                pltpu.make_async_copy(pos8.at[idx_b[b]], rows_b[b],
                                      gsem[b]).wait()
                pltpu.make_async_copy(
                    pos8.at[pl.ds(0, _C)], ctr_b[b], gsem[b]).wait()

                @pl.when(c2 < n_full)
                def _():
                    base2 = pl.multiple_of(c2 * CK, 8)
                    pltpu.async_copy(nbrf.at[pl.ds(base2, CK)], idx_b[b],
                                     isem[b])

                @pl.when(cn < n_full)
                def _():
                    basen = pl.multiple_of(cn * CK, 8)
                    pltpu.make_async_copy(nbrf.at[pl.ds(basen, CK)],
                                          idx_b[nb], isem[nb]).wait()
                    pltpu.async_copy(pos8.at[idx_b[nb]], rows_b[nb], gsem[nb])
                    pltpu.async_copy(
                        pos8.at[pl.ds(pl.multiple_of(cn * _C, _C), _C)],
                        ctr_b[nb], gsem[nb])

                compute_chunk(rows_b[b], ctr_b[b],
                              pl.multiple_of(c * _C, _C), _C)

        return carry

    lax.fori_loop(0, tt_max, tt_body, 0)

    if tail_nodes:
        @pl.when(wid == n_full % _NW)
        def _():
            node_base = pl.multiple_of(n_full * _C, _C)
            n_e = tail_nodes * K
            ebase = pl.multiple_of(node_base * K, 8)
            pltpu.sync_copy(nbrf.at[pl.ds(ebase, n_e)],
                            idx_v0.at[pl.ds(0, n_e)])
            pltpu.sync_copy(pos8.at[pl.ds(node_base, tail_nodes)],
                            ctr_v0.at[pl.ds(0, tail_nodes)])
            pltpu.async_copy(pos8.at[idx_v0.at[pl.ds(0, n_e)]],
                             rows_v0.at[pl.ds(0, n_e)], tsem).wait()
            compute_chunk(rows_v0, ctr_v0, node_base, tail_nodes)


@functools.cache
def _build(nodes, K):
    n_full = nodes // _C
    tail_nodes = nodes - n_full * _C
    t_max = -(-n_full // _NW)
    tt_max = (t_max + 1) // 2
    body = functools.partial(_sc_body, K, n_full, tail_nodes, tt_max)
    mesh = plsc.VectorSubcoreMesh(core_axis_name="c", subcore_axis_name="s",
                                  num_cores=_NC, num_subcores=_NS)
    fn = pl.kernel(
        body,
        out_type=(
            jax.ShapeDtypeStruct((nodes, _N_BASIS), jnp.float32),
            jax.ShapeDtypeStruct((nodes, 3 * _N_BASIS), jnp.float32),
        ),
        mesh=mesh,
        scratch_types=[
            pltpu.VMEM((_C * K,), jnp.int32),
            pltpu.VMEM((_C * K,), jnp.int32),
            pltpu.VMEM((_C * K, 8), jnp.float32),
            pltpu.VMEM((_C * K, 8), jnp.float32),
            pltpu.VMEM((_C, 8), jnp.float32),
            pltpu.VMEM((_C, 8), jnp.float32),
            pltpu.VMEM((8, _L), jnp.float32),
            pltpu.VMEM((_C, _N_BASIS), jnp.float32),
            pltpu.VMEM((_C, 3 * _N_BASIS), jnp.float32),
            pltpu.VMEM((_N_BASIS + 3, K, _L), jnp.float32),
            pltpu.SemaphoreType.DMA,
            pltpu.SemaphoreType.DMA,
            pltpu.SemaphoreType.DMA,
            pltpu.SemaphoreType.DMA,
            pltpu.SemaphoreType.DMA,
        ],
        compiler_params=pltpu.CompilerParams(needs_layout_passes=False,
                                             use_tc_tiling_on_sc=False),
    )
    return fn


def kernel(pos, box, neighbors):
    batch, nodes, _ = pos.shape
    K = neighbors.shape[1]
    fn = _build(nodes, K)
    p = pos.reshape(nodes, 3)
    pos8 = jnp.pad(p, ((0, 0), (0, 5)))
    nbrf = neighbors.reshape(-1)
    inv_box = (1.0 / box).astype(jnp.float32)
    ang = (box / (2.0 * math.pi)).astype(jnp.float32)
    params = jnp.concatenate([
        jnp.repeat(inv_box[:, None], _L, axis=1),
        jnp.repeat(ang[:, None], _L, axis=1),
        jnp.zeros((2, _L), jnp.float32),
    ], axis=0)
    y_flat, yv_flat = fn(pos8, nbrf, params)
    y = y_flat.reshape(batch, nodes, _N_BASIS)
    yv = yv_flat.reshape(batch, nodes, _N_BASIS, 3)
    return (y, yv)
